# baseline (speedup 1.0000x reference)
import jax
import jax.numpy as jnp
from jax.experimental import pallas as pl
from jax.experimental.pallas import tpu as pltpu

_TB = 512
_NSPLIT = 8


def _fused_kernel(*refs):
    x_refs = refs[:_NSPLIT]
    wg_ref, bg_ref, wo_ref, bo_ref, o_ref = refs[_NSPLIT:]
    tb, n_exp, d = x_refs[0].shape

    wg = wg_ref[...]
    wo = wo_ref[...]
    bg = bg_ref[...]
    bo = bo_ref[...]

    for j in range(_NSPLIT):
        x = x_refs[j][...].reshape(tb, n_exp * d)
        xb = x.astype(jnp.bfloat16)

        logits = jnp.dot(xb, wg, preferred_element_type=jnp.float32) + bg
        logits = logits - jnp.max(logits, axis=-1, keepdims=True)
        p = jnp.exp(logits)
        w = p / jnp.sum(p, axis=-1, keepdims=True)

        pooled = w[:, 0:1] * x[:, 0:d]
        for e in range(1, n_exp):
            pooled = pooled + w[:, e:e + 1] * x[:, e * d:(e + 1) * d]

        out = jnp.dot(pooled.astype(jnp.bfloat16), wo,
                      preferred_element_type=jnp.float32)
        o_ref[j * tb:(j + 1) * tb, :] = out + bo


def kernel(x, wg, bg, wo, bo):
    B, E, D = x.shape
    N = wo.shape[1]

    wg_b = wg.astype(jnp.bfloat16)
    wo_b = wo.astype(jnp.bfloat16)
    bg2 = bg.reshape(1, E).astype(jnp.float32)
    bo2 = bo.reshape(1, N).astype(jnp.float32)

    step = _TB * _NSPLIT
    tb = _TB if B % step == 0 else 8
    grid = (B // (tb * _NSPLIT),)

    def x_spec(j):
        return pl.BlockSpec((tb, E, D), lambda i, j=j: (i * _NSPLIT + j, 0, 0))

    out = pl.pallas_call(
        _fused_kernel,
        out_shape=jax.ShapeDtypeStruct((B, N), jnp.float32),
        grid=grid,
        in_specs=[x_spec(j) for j in range(_NSPLIT)] + [
            pl.BlockSpec((E * D, E), lambda i: (0, 0)),
            pl.BlockSpec((1, E), lambda i: (0, 0)),
            pl.BlockSpec((D, N), lambda i: (0, 0)),
            pl.BlockSpec((1, N), lambda i: (0, 0)),
        ],
        out_specs=pl.BlockSpec((tb * _NSPLIT, N), lambda i: (i, 0)),
        compiler_params=pltpu.CompilerParams(
            dimension_semantics=("parallel",),
        ),
    )(*([x] * _NSPLIT), wg_b, bg2, wo_b, bo2)
    return out.astype(x.dtype)


# speedup vs baseline: 1.1083x; 1.1083x over previous
import jax
import jax.numpy as jnp
from jax.experimental import pallas as pl
from jax.experimental.pallas import tpu as pltpu

_TB = 256
_NSPLIT = 4


def _fused_kernel(*refs):
    x_refs = refs[:_NSPLIT]
    wg_ref, bg_ref, wo_ref, bo_ref, o_ref = refs[_NSPLIT:]
    tb, n_exp, d = x_refs[0].shape

    wg = wg_ref[...]
    wo = wo_ref[...]
    bg = bg_ref[...]
    bo = bo_ref[...]

    for j in range(_NSPLIT):
        x = x_refs[j][...].reshape(tb, n_exp * d)
        xb = x.astype(jnp.bfloat16)

        logits = jnp.dot(xb, wg, preferred_element_type=jnp.float32) + bg
        logits = logits - jnp.max(logits, axis=-1, keepdims=True)
        p = jnp.exp(logits)
        w = p / jnp.sum(p, axis=-1, keepdims=True)

        pooled = w[:, 0:1] * x[:, 0:d]
        for e in range(1, n_exp):
            pooled = pooled + w[:, e:e + 1] * x[:, e * d:(e + 1) * d]

        out = jnp.dot(pooled.astype(jnp.bfloat16), wo,
                      preferred_element_type=jnp.float32)
        o_ref[j * tb:(j + 1) * tb, :] = out + bo


def kernel(x, wg, bg, wo, bo):
    B, E, D = x.shape
    N = wo.shape[1]

    wg_b = wg.astype(jnp.bfloat16)
    wo_b = wo.astype(jnp.bfloat16)
    bg2 = bg.reshape(1, E).astype(jnp.float32)
    bo2 = bo.reshape(1, N).astype(jnp.float32)

    step = _TB * _NSPLIT
    tb = _TB if B % step == 0 else 8
    grid = (B // (tb * _NSPLIT),)

    def x_spec(j):
        return pl.BlockSpec((tb, E, D), lambda i, j=j: (i * _NSPLIT + j, 0, 0))

    out = pl.pallas_call(
        _fused_kernel,
        out_shape=jax.ShapeDtypeStruct((B, N), jnp.float32),
        grid=grid,
        in_specs=[x_spec(j) for j in range(_NSPLIT)] + [
            pl.BlockSpec((E * D, E), lambda i: (0, 0)),
            pl.BlockSpec((1, E), lambda i: (0, 0)),
            pl.BlockSpec((D, N), lambda i: (0, 0)),
            pl.BlockSpec((1, N), lambda i: (0, 0)),
        ],
        out_specs=pl.BlockSpec((tb * _NSPLIT, N), lambda i: (i, 0)),
        compiler_params=pltpu.CompilerParams(
            dimension_semantics=("parallel",),
        ),
    )(*([x] * _NSPLIT), wg_b, bg2, wo_b, bo2)
    return out.astype(x.dtype)


# speedup vs baseline: 1.1115x; 1.0029x over previous
import jax
import jax.numpy as jnp
from jax.experimental import pallas as pl
from jax.experimental.pallas import tpu as pltpu

_TB = 256
_NSPLIT = 4


def _fused_kernel(*refs):
    x_refs = refs[:_NSPLIT]
    wg_ref, bg_ref, wo_ref, bo_ref, o_ref = refs[_NSPLIT:]
    tb, n_exp, d = x_refs[0].shape

    wg = wg_ref[...]
    wo = wo_ref[...]
    bg = bg_ref[...]
    bo = bo_ref[...]

    for j in range(_NSPLIT):
        xb = x_refs[j][...].astype(jnp.bfloat16).reshape(tb, n_exp * d)

        logits = jnp.dot(xb, wg, preferred_element_type=jnp.float32) + bg
        logits = logits - jnp.max(logits, axis=-1, keepdims=True)
        p = jnp.exp(logits)
        w = p / jnp.sum(p, axis=-1, keepdims=True)

        pooled = w[:, 0:1] * xb[:, 0:d]
        for e in range(1, n_exp):
            pooled = pooled + w[:, e:e + 1] * xb[:, e * d:(e + 1) * d]

        out = jnp.dot(pooled.astype(jnp.bfloat16), wo,
                      preferred_element_type=jnp.float32)
        o_ref[j * tb:(j + 1) * tb, :] = out + bo


def kernel(x, wg, bg, wo, bo):
    B, E, D = x.shape
    N = wo.shape[1]

    wg_b = wg.astype(jnp.bfloat16)
    wo_b = wo.astype(jnp.bfloat16)
    bg2 = bg.reshape(1, E).astype(jnp.float32)
    bo2 = bo.reshape(1, N).astype(jnp.float32)

    step = _TB * _NSPLIT
    tb = _TB if B % step == 0 else 8
    grid = (B // (tb * _NSPLIT),)

    def x_spec(j):
        return pl.BlockSpec((tb, E, D), lambda i, j=j: (i * _NSPLIT + j, 0, 0))

    out = pl.pallas_call(
        _fused_kernel,
        out_shape=jax.ShapeDtypeStruct((B, N), jnp.float32),
        grid=grid,
        in_specs=[x_spec(j) for j in range(_NSPLIT)] + [
            pl.BlockSpec((E * D, E), lambda i: (0, 0)),
            pl.BlockSpec((1, E), lambda i: (0, 0)),
            pl.BlockSpec((D, N), lambda i: (0, 0)),
            pl.BlockSpec((1, N), lambda i: (0, 0)),
        ],
        out_specs=pl.BlockSpec((tb * _NSPLIT, N), lambda i: (i, 0)),
        compiler_params=pltpu.CompilerParams(
            dimension_semantics=("parallel",),
        ),
    )(*([x] * _NSPLIT), wg_b, bg2, wo_b, bo2)
    return out.astype(x.dtype)
